# Initial kernel scaffold; baseline (speedup 1.0000x reference)
import functools

import jax
import jax.numpy as jnp
from jax import lax
from jax.experimental import pallas as pl
from jax.experimental.pallas import tpu as pltpu
from jax.experimental.pallas import tpu_sc as plsc

N = 10000
E = 320000
NPAD = 10112
NBLK = 79
NC, NS, L = 2, 16, 16
NW = NC * NS
EPT = E // NW
CPT = NPAD // 128
STRIPE = NPAD // NS


def _wid():
    return lax.axis_index("s") * NC + lax.axis_index("c")


def _zero_vmem_2d(buf, nrows):
    z = jnp.zeros((L,), jnp.float32)

    def body(i, _):
        for k in range(128 // L):
            buf[i, pl.ds(k * L, L)] = z
        return 0

    lax.fori_loop(0, nrows, body, 0)


def _zero_vmem_1d(buf, n):
    z = jnp.zeros((L,), jnp.float32)

    def body(i, _):
        buf[pl.ds(i * L, L)] = z
        return 0

    lax.fori_loop(0, n // L, body, 0)


def _zero_stripe(S_spmem, zbuf):
    s = lax.axis_index("s")
    _zero_vmem_2d(zbuf, NBLK)
    for k in range(STRIPE // NBLK):
        pltpu.sync_copy(zbuf, S_spmem.at[pl.ds(s * STRIPE + k * NBLK, NBLK)])


def _sc_deg_body(dstp_hbm, out_hbm, dst_v, hist_v):
    w = _wid()
    pltpu.sync_copy(dstp_hbm.at[w], dst_v)
    _zero_vmem_1d(hist_v, NPAD)
    ones = jnp.ones((L,), jnp.float32)

    def blk(j, _):
        for k in range(128 // L):
            idx = dst_v[j, pl.ds(k * L, L)]
            plsc.addupdate_scatter(hist_v, [idx], ones)
        return 0

    lax.fori_loop(0, CPT, blk, 0)
    pltpu.sync_copy(hist_v, out_hbm.at[w])


def _sc_gcn_body(hs_hbm, srcp_hbm, dstp_hbm, out_hbm,
                 src_v, dst_v, rows_v, zbuf, S_spmem, sem):
    c = lax.axis_index("c")
    w = _wid()
    _zero_stripe(S_spmem, zbuf)
    pltpu.sync_copy(srcp_hbm.at[w], src_v)
    pltpu.sync_copy(dstp_hbm.at[w], dst_v)
    plsc.subcore_barrier()

    def chunk(j, _):
        pltpu.async_copy(hs_hbm.at[src_v.at[j]], rows_v, sem).wait()
        pltpu.sync_copy(rows_v, S_spmem.at[dst_v.at[j]], add=True)
        return 0

    lax.fori_loop(0, CPT, chunk, 0)
    plsc.subcore_barrier()
    s = lax.axis_index("s")
    pltpu.sync_copy(S_spmem.at[pl.ds(s * STRIPE, STRIPE)],
                    out_hbm.at[c, pl.ds(s * STRIPE, STRIPE)])


def _sc_amax_body(asrc_hbm, adst_hbm, srcp_hbm, dstp_hbm, out_hbm,
                  src_v, dst_v, asrc_v, adst_v, mbuf_v):
    w = _wid()
    pltpu.sync_copy(srcp_hbm.at[w], src_v)
    pltpu.sync_copy(dstp_hbm.at[w], dst_v)
    pltpu.sync_copy(asrc_hbm, asrc_v)
    pltpu.sync_copy(adst_hbm, adst_v)

    def blk(j, m):
        for k in range(128 // L):
            s16 = src_v[j, pl.ds(k * L, L)]
            d16 = dst_v[j, pl.ds(k * L, L)]
            z = plsc.load_gather(asrc_v, [s16]) + plsc.load_gather(adst_v, [d16])
            m = jnp.maximum(m, jnp.maximum(z, 0.2 * z))
        return m

    m = lax.fori_loop(0, CPT, blk, jnp.full((L,), -3e38, jnp.float32))

    def selfblk(i, m):
        z = asrc_v[pl.ds(i * L, L)] + adst_v[pl.ds(i * L, L)]
        return jnp.maximum(m, jnp.maximum(z, 0.2 * z))

    m = lax.fori_loop(0, NPAD // L, selfblk, m)
    mbuf_v[...] = m
    pltpu.sync_copy(mbuf_v, out_hbm.at[w])


def _sc_gat_body(h2_hbm, asrc_hbm, adst_hbm, srcp_hbm, dstp_hbm, maxes_hbm,
                 outS_hbm, outD_hbm,
                 src_v, dst_v, asrc_v, adst_v, rows_v, ex_v, hist_v, max_v,
                 S_spmem, sem):
    c = lax.axis_index("c")
    s = lax.axis_index("s")
    w = _wid()
    _zero_stripe(S_spmem, rows_v)
    pltpu.sync_copy(srcp_hbm.at[w], src_v)
    pltpu.sync_copy(dstp_hbm.at[w], dst_v)
    pltpu.sync_copy(asrc_hbm, asrc_v)
    pltpu.sync_copy(adst_hbm, adst_v)
    pltpu.sync_copy(maxes_hbm, max_v)
    _zero_vmem_1d(hist_v, NPAD)

    m = jnp.full((L,), -3e38, jnp.float32)
    for i in range(NW):
        m = jnp.maximum(m, max_v[i])
    C = jnp.full((L,), lax.reduce_max(m, axes=(0,)), jnp.float32)
    plsc.subcore_barrier()

    def chunk(j, _):
        pltpu.async_copy(h2_hbm.at[src_v.at[j]], rows_v, sem).wait()
        for k in range(128 // L):
            s16 = src_v[j, pl.ds(k * L, L)]
            d16 = dst_v[j, pl.ds(k * L, L)]
            z = plsc.load_gather(asrc_v, [s16]) + plsc.load_gather(adst_v, [d16])
            ex = jnp.exp(jnp.maximum(z, 0.2 * z) - C)
            plsc.addupdate_scatter(hist_v, [d16], ex)
            ex_v[pl.ds(k * L, L)] = ex

        def scale(e, _):
            sp = plsc.load_gather(ex_v, [jnp.zeros((L,), jnp.int32) + e])
            for k in range(128 // L):
                rows_v[e, pl.ds(k * L, L)] = rows_v[e, pl.ds(k * L, L)] * sp
            return 0

        lax.fori_loop(0, 128, scale, 0)
        pltpu.sync_copy(rows_v, S_spmem.at[dst_v.at[j]], add=True)
        return 0

    lax.fori_loop(0, CPT, chunk, 0)
    plsc.subcore_barrier()
    pltpu.sync_copy(S_spmem.at[pl.ds(s * STRIPE, STRIPE)],
                    outS_hbm.at[c, pl.ds(s * STRIPE, STRIPE)])
    pltpu.sync_copy(hist_v, outD_hbm.at[w])


_MESH = plsc.VectorSubcoreMesh(core_axis_name="c", subcore_axis_name="s")


def _sc_deg(dstp):
    return pl.kernel(
        _sc_deg_body,
        out_type=jax.ShapeDtypeStruct((NW, NPAD), jnp.float32),
        mesh=_MESH,
        scratch_types=[
            pltpu.VMEM((NBLK, 128), jnp.int32),
            pltpu.VMEM((NPAD,), jnp.float32),
        ],
    )(dstp)


def _sc_gcn(hs, srcp, dstp, F):
    return pl.kernel(
        _sc_gcn_body,
        out_type=jax.ShapeDtypeStruct((NC, NPAD, F), jnp.float32),
        mesh=_MESH,
        scratch_types=[
            pltpu.VMEM((NBLK, 128), jnp.int32),
            pltpu.VMEM((NBLK, 128), jnp.int32),
            pltpu.VMEM((128, F), jnp.float32),
            pltpu.VMEM((NBLK, 128), jnp.float32),
            pltpu.VMEM_SHARED((NPAD, F), jnp.float32),
            pltpu.SemaphoreType.DMA,
        ],
    )(hs, srcp, dstp)


def _sc_amax(asrc, adst, srcp, dstp):
    return pl.kernel(
        _sc_amax_body,
        out_type=jax.ShapeDtypeStruct((NW, L), jnp.float32),
        mesh=_MESH,
        scratch_types=[
            pltpu.VMEM((NBLK, 128), jnp.int32),
            pltpu.VMEM((NBLK, 128), jnp.int32),
            pltpu.VMEM((NPAD,), jnp.float32),
            pltpu.VMEM((NPAD,), jnp.float32),
            pltpu.VMEM((L,), jnp.float32),
        ],
    )(asrc, adst, srcp, dstp)


def _sc_gat(h2, asrc, adst, srcp, dstp, maxes):
    return pl.kernel(
        _sc_gat_body,
        out_type=(jax.ShapeDtypeStruct((NC, NPAD, 128), jnp.float32),
                  jax.ShapeDtypeStruct((NW, NPAD), jnp.float32)),
        mesh=_MESH,
        scratch_types=[
            pltpu.VMEM((NBLK, 128), jnp.int32),
            pltpu.VMEM((NBLK, 128), jnp.int32),
            pltpu.VMEM((NPAD,), jnp.float32),
            pltpu.VMEM((NPAD,), jnp.float32),
            pltpu.VMEM((128, 128), jnp.float32),
            pltpu.VMEM((128,), jnp.float32),
            pltpu.VMEM((NPAD,), jnp.float32),
            pltpu.VMEM((NW, L), jnp.float32),
            pltpu.VMEM_SHARED((NPAD, 128), jnp.float32),
            pltpu.SemaphoreType.DMA,
        ],
    )(h2, asrc, adst, srcp, dstp, maxes)


def _tc_a_body(x_ref, w1_ref, degp_ref, hs_ref, dinv_ref):
    deg = jnp.sum(degp_ref[...], axis=0) + 1.0
    dinv = lax.rsqrt(deg)
    h = jnp.dot(x_ref[...], w1_ref[...], preferred_element_type=jnp.float32)
    hs_ref[...] = h * dinv[None, :].reshape(128, 1)
    dinv_ref[...] = dinv[None, :]


def _tc_a(x_pad, W1, deg_parts):
    return pl.pallas_call(
        _tc_a_body,
        grid=(NBLK,),
        in_specs=[
            pl.BlockSpec((128, 192), lambda i: (i, 0)),
            pl.BlockSpec((192, 128), lambda i: (0, 0)),
            pl.BlockSpec((NW, 128), lambda i: (0, i)),
        ],
        out_specs=[
            pl.BlockSpec((128, 128), lambda i: (i, 0)),
            pl.BlockSpec((1, 128), lambda i: (i, 0)),
        ],
        out_shape=[
            jax.ShapeDtypeStruct((NPAD, 128), jnp.float32),
            jax.ShapeDtypeStruct((NBLK, 128), jnp.float32),
        ],
    )(x_pad, W1, deg_parts)


def _tc_b_body(s1p_ref, hs1_ref, dinv_ref, b1_ref, wg_ref, asv_ref, adv_ref,
               h2_ref, asrc_ref, adst_ref):
    dv = dinv_ref[...].reshape(128, 1)
    g1 = dv * (s1p_ref[0] + s1p_ref[1] + hs1_ref[...]) + b1_ref[...]
    h2 = jnp.dot(g1, wg_ref[...], preferred_element_type=jnp.float32)
    h2_ref[...] = h2
    asrc_ref[...] = jnp.sum(h2 * asv_ref[...], axis=1)[None, :]
    adst_ref[...] = jnp.sum(h2 * adv_ref[...], axis=1)[None, :]


def _tc_b(S1p, hs1, dinv2d, b1, Wg, att_src, att_dst):
    return pl.pallas_call(
        _tc_b_body,
        grid=(NBLK,),
        in_specs=[
            pl.BlockSpec((2, 128, 128), lambda i: (0, i, 0)),
            pl.BlockSpec((128, 128), lambda i: (i, 0)),
            pl.BlockSpec((1, 128), lambda i: (i, 0)),
            pl.BlockSpec((1, 128), lambda i: (0, 0)),
            pl.BlockSpec((128, 128), lambda i: (0, 0)),
            pl.BlockSpec((1, 128), lambda i: (0, 0)),
            pl.BlockSpec((1, 128), lambda i: (0, 0)),
        ],
        out_specs=[
            pl.BlockSpec((128, 128), lambda i: (i, 0)),
            pl.BlockSpec((1, 128), lambda i: (i, 0)),
            pl.BlockSpec((1, 128), lambda i: (i, 0)),
        ],
        out_shape=[
            jax.ShapeDtypeStruct((NPAD, 128), jnp.float32),
            jax.ShapeDtypeStruct((NBLK, 128), jnp.float32),
            jax.ShapeDtypeStruct((NBLK, 128), jnp.float32),
        ],
    )(S1p, hs1, dinv2d, b1, Wg, att_src, att_dst)


def _tc_c_body(s2p_ref, denp_ref, h2_ref, asrc_ref, adst_ref, maxes_ref,
               dinv_ref, bg_ref, w2_ref, hs3_ref):
    C = jnp.max(maxes_ref[...])
    z = asrc_ref[...] + adst_ref[...]
    es = jnp.exp(jnp.maximum(z, 0.2 * z) - C).reshape(128, 1)
    Stot = s2p_ref[0] + s2p_ref[1] + es * h2_ref[...]
    den = jnp.sum(denp_ref[...], axis=0)[None, :].reshape(128, 1) + es
    out2 = jnp.maximum(Stot / (den + 1e-16) + bg_ref[...], 0.0)
    h3 = jnp.dot(out2, w2_ref[...], preferred_element_type=jnp.float32)
    hs3_ref[...] = h3 * dinv_ref[...].reshape(128, 1)


def _tc_c(S2p, den_parts, h2, asrc2d, adst2d, maxes, dinv2d, bg, W2):
    return pl.pallas_call(
        _tc_c_body,
        grid=(NBLK,),
        in_specs=[
            pl.BlockSpec((2, 128, 128), lambda i: (0, i, 0)),
            pl.BlockSpec((NW, 128), lambda i: (0, i)),
            pl.BlockSpec((128, 128), lambda i: (i, 0)),
            pl.BlockSpec((1, 128), lambda i: (i, 0)),
            pl.BlockSpec((1, 128), lambda i: (i, 0)),
            pl.BlockSpec((NW, L), lambda i: (0, 0)),
            pl.BlockSpec((1, 128), lambda i: (i, 0)),
            pl.BlockSpec((1, 128), lambda i: (0, 0)),
            pl.BlockSpec((128, 64), lambda i: (0, 0)),
        ],
        out_specs=[pl.BlockSpec((128, 64), lambda i: (i, 0))],
        out_shape=[jax.ShapeDtypeStruct((NPAD, 64), jnp.float32)],
    )(S2p, den_parts, h2, asrc2d, adst2d, maxes, dinv2d, bg, W2)


def _tc_d_body(s3p_ref, hs3_ref, dinv_ref, b2_ref, wf_ref, bf_ref, out_ref):
    dv = dinv_ref[...].reshape(128, 1)
    g3 = dv * (s3p_ref[0] + s3p_ref[1] + hs3_ref[...]) + b2_ref[...]
    out_ref[...] = (jnp.dot(g3, wf_ref[...], preferred_element_type=jnp.float32)
                    + bf_ref[...])


def _tc_d(S3p, hs3, dinv2d, b2, Wf, bf):
    return pl.pallas_call(
        _tc_d_body,
        grid=(NBLK,),
        in_specs=[
            pl.BlockSpec((2, 128, 64), lambda i: (0, i, 0)),
            pl.BlockSpec((128, 64), lambda i: (i, 0)),
            pl.BlockSpec((1, 128), lambda i: (i, 0)),
            pl.BlockSpec((1, 64), lambda i: (0, 0)),
            pl.BlockSpec((64, 192), lambda i: (0, 0)),
            pl.BlockSpec((1, 192), lambda i: (0, 0)),
        ],
        out_specs=[pl.BlockSpec((128, 192), lambda i: (i, 0))],
        out_shape=[jax.ShapeDtypeStruct((NPAD, 192), jnp.float32)],
    )(S3p, hs3, dinv2d, b2, Wf, bf)


def kernel(x, edge_index, W1, b1, Wg, att_src, att_dst, bg, W2, b2, Wf, bf):
    src = edge_index[0].astype(jnp.int32)
    dst = edge_index[1].astype(jnp.int32)
    pad_cfg = ((0, 0), (0, NPAD - EPT))
    srcp = jnp.pad(src.reshape(NW, EPT), pad_cfg, constant_values=N).reshape(NW, NBLK, 128)
    dstp = jnp.pad(dst.reshape(NW, EPT), pad_cfg, constant_values=N).reshape(NW, NBLK, 128)
    x_pad = jnp.pad(x, ((0, NPAD - N), (0, 0)))

    deg_parts = _sc_deg(dstp)
    hs1, dinv2d = _tc_a(x_pad, W1, deg_parts)
    S1p = _sc_gcn(hs1, srcp, dstp, 128)
    h2, asrc2d, adst2d = _tc_b(S1p, hs1, dinv2d, b1.reshape(1, 128), Wg,
                               att_src.reshape(1, 128), att_dst.reshape(1, 128))
    asrc = asrc2d.reshape(NPAD)
    adst = adst2d.reshape(NPAD)
    maxes = _sc_amax(asrc, adst, srcp, dstp)
    S2p, den_parts = _sc_gat(h2, asrc, adst, srcp, dstp, maxes)
    hs3, = _tc_c(S2p, den_parts, h2, asrc2d, adst2d, maxes, dinv2d,
                 bg.reshape(1, 128), W2)
    S3p = _sc_gcn(hs3, srcp, dstp, 64)
    out, = _tc_d(S3p, hs3, dinv2d, b2.reshape(1, 64), Wf, bf.reshape(1, 192))
    return out[:N]


# speedup vs baseline: 14.4092x; 14.4092x over previous
import jax
import jax.numpy as jnp
from jax import lax
from jax.experimental import pallas as pl
from jax.experimental.pallas import tpu as pltpu
from jax.experimental.pallas import tpu_sc as plsc

N = 10000
E = 320000
NPAD = 10112
NBLK = 79
NC, NS, L = 2, 16, 16
NW = NC * NS
EPT = E // NW
CPT = NPAD // 128
EPT2 = E // NS
CPT2 = 158
EPAD2 = CPT2 * 128
STRIPE = NPAD // NS


def _wid():
    return lax.axis_index("s") * NC + lax.axis_index("c")


def _zero_vmem_2d(buf, nrows, ncols):
    z = jnp.zeros((L,), jnp.float32)

    def body(i, _):
        for k in range(ncols // L):
            buf[i, pl.ds(k * L, L)] = z
        return 0

    lax.fori_loop(0, nrows, body, 0)


def _zero_vmem_1d(buf, n):
    z = jnp.zeros((L,), jnp.float32)

    def body(i, _):
        buf[pl.ds(i * L, L)] = z
        return 0

    lax.fori_loop(0, n // L, body, 0)


def _zero_stripe(S_spmem, zbuf, F2):
    s = lax.axis_index("s")
    _zero_vmem_2d(zbuf, NBLK, F2)
    for k in range(STRIPE // NBLK):
        pltpu.sync_copy(zbuf, S_spmem.at[pl.ds(s * STRIPE + k * NBLK, NBLK)])


def _sc_deg_body(dstp_hbm, out_hbm, dst_v, hist_v):
    w = _wid()
    pltpu.sync_copy(dstp_hbm.at[w], dst_v)
    _zero_vmem_1d(hist_v, NPAD)
    ones = jnp.ones((L,), jnp.float32)

    def blk(j, _):
        for k in range(128 // L):
            idx = dst_v[j, pl.ds(k * L, L)]
            plsc.addupdate_scatter(hist_v, [idx], ones)
        return 0

    lax.fori_loop(0, CPT, blk, 0)
    pltpu.sync_copy(hist_v, out_hbm.at[w])


def _sc_gcn_body(hs_hbm, srcp_hbm, dstp_hbm, out_hbm,
                 src_v, dst_v, rows_v, zbuf, S_spmem, sem):
    c = lax.axis_index("c")
    s = lax.axis_index("s")
    F2 = rows_v.shape[1]
    _zero_stripe(S_spmem, zbuf, F2)
    pltpu.sync_copy(srcp_hbm.at[c, s], src_v)
    pltpu.sync_copy(dstp_hbm.at[s], dst_v)
    plsc.subcore_barrier()

    def chunk(j, _):
        pltpu.async_copy(hs_hbm.at[src_v.at[j]], rows_v, sem).wait()
        pltpu.sync_copy(rows_v, S_spmem.at[dst_v.at[j]], add=True)
        return 0

    lax.fori_loop(0, CPT2, chunk, 0)
    plsc.subcore_barrier()
    pltpu.sync_copy(S_spmem.at[pl.ds(s * STRIPE, STRIPE)],
                    out_hbm.at[c, pl.ds(s * STRIPE, STRIPE)])


def _sc_amax_body(asrc_hbm, adst_hbm, srcp_hbm, dstp_hbm, out_hbm,
                  src_v, dst_v, asrc_v, adst_v, mbuf_v):
    w = _wid()
    pltpu.sync_copy(srcp_hbm.at[w], src_v)
    pltpu.sync_copy(dstp_hbm.at[w], dst_v)
    pltpu.sync_copy(asrc_hbm, asrc_v)
    pltpu.sync_copy(adst_hbm, adst_v)

    def blk(j, m):
        for k in range(128 // L):
            s16 = src_v[j, pl.ds(k * L, L)]
            d16 = dst_v[j, pl.ds(k * L, L)]
            z = plsc.load_gather(asrc_v, [s16]) + plsc.load_gather(adst_v, [d16])
            m = jnp.maximum(m, jnp.maximum(z, 0.2 * z))
        return m

    m = lax.fori_loop(0, CPT, blk, jnp.full((L,), -3e38, jnp.float32))

    def selfblk(i, m):
        z = asrc_v[pl.ds(i * L, L)] + adst_v[pl.ds(i * L, L)]
        return jnp.maximum(m, jnp.maximum(z, 0.2 * z))

    m = lax.fori_loop(0, NPAD // L, selfblk, m)
    mbuf_v[...] = m
    pltpu.sync_copy(mbuf_v, out_hbm.at[w])


def _sc_gat_body(h2_hbm, asrc_hbm, adst_hbm, srcp_hbm, dstp_hbm, maxes_hbm,
                 outS_hbm, outD_hbm,
                 src_v, dst_v, asrc_v, adst_v, rows_v, zbuf, ex_v, hist_v,
                 max_v, S_spmem, sem):
    c = lax.axis_index("c")
    s = lax.axis_index("s")
    F2 = rows_v.shape[1]
    _zero_stripe(S_spmem, zbuf, F2)
    pltpu.sync_copy(srcp_hbm.at[c, s], src_v)
    pltpu.sync_copy(dstp_hbm.at[s], dst_v)
    pltpu.sync_copy(asrc_hbm, asrc_v)
    pltpu.sync_copy(adst_hbm, adst_v)
    pltpu.sync_copy(maxes_hbm, max_v)
    _zero_vmem_1d(hist_v, NPAD)

    m = jnp.full((L,), -3e38, jnp.float32)
    for i in range(NW):
        m = jnp.maximum(m, max_v[i])
    C = jnp.full((L,), lax.reduce_max(m, axes=(0,)), jnp.float32)
    off = jnp.full((L,), NPAD, jnp.int32) * c
    plsc.subcore_barrier()

    def chunk(j, _):
        pltpu.async_copy(h2_hbm.at[src_v.at[j]], rows_v, sem).wait()
        for k in range(128 // L):
            s16 = src_v[j, pl.ds(k * L, L)] - off
            d16 = dst_v[j, pl.ds(k * L, L)]
            z = plsc.load_gather(asrc_v, [s16]) + plsc.load_gather(adst_v, [d16])
            ex = jnp.exp(jnp.maximum(z, 0.2 * z) - C)
            plsc.addupdate_scatter(hist_v, [d16], ex)
            ex_v[pl.ds(k * L, L)] = ex

        def scale(e, _):
            sp = plsc.load_gather(ex_v, [jnp.zeros((L,), jnp.int32) + e])
            for k in range(F2 // L):
                rows_v[e, pl.ds(k * L, L)] = rows_v[e, pl.ds(k * L, L)] * sp
            return 0

        lax.fori_loop(0, 128, scale, 0)
        pltpu.sync_copy(rows_v, S_spmem.at[dst_v.at[j]], add=True)
        return 0

    lax.fori_loop(0, CPT2, chunk, 0)
    plsc.subcore_barrier()
    pltpu.sync_copy(S_spmem.at[pl.ds(s * STRIPE, STRIPE)],
                    outS_hbm.at[c, pl.ds(s * STRIPE, STRIPE)])
    pltpu.sync_copy(hist_v, outD_hbm.at[c, s])


_MESH = plsc.VectorSubcoreMesh(core_axis_name="c", subcore_axis_name="s")
_SC_PARAMS = pltpu.CompilerParams(needs_layout_passes=False, use_tc_tiling_on_sc=False)


def _sc_deg(dstp):
    return pl.kernel(
        _sc_deg_body,
        compiler_params=_SC_PARAMS,
        out_type=jax.ShapeDtypeStruct((NW, NPAD), jnp.float32),
        mesh=_MESH,
        scratch_types=[
            pltpu.VMEM((CPT, 128), jnp.int32),
            pltpu.VMEM((NPAD,), jnp.float32),
        ],
    )(dstp)


def _sc_gcn(hs_split, srcp_off, dstp, F2):
    return pl.kernel(
        _sc_gcn_body,
        compiler_params=_SC_PARAMS,
        out_type=jax.ShapeDtypeStruct((NC, NPAD, F2), jnp.float32),
        mesh=_MESH,
        scratch_types=[
            pltpu.VMEM((CPT2, 128), jnp.int32),
            pltpu.VMEM((CPT2, 128), jnp.int32),
            pltpu.VMEM((128, F2), jnp.float32),
            pltpu.VMEM((NBLK, F2), jnp.float32),
            pltpu.VMEM_SHARED((NPAD, F2), jnp.float32),
            pltpu.SemaphoreType.DMA,
        ],
    )(hs_split, srcp_off, dstp)


def _sc_amax(asrc, adst, srcp32, dstp32):
    return pl.kernel(
        _sc_amax_body,
        compiler_params=_SC_PARAMS,
        out_type=jax.ShapeDtypeStruct((NW, L), jnp.float32),
        mesh=_MESH,
        scratch_types=[
            pltpu.VMEM((CPT, 128), jnp.int32),
            pltpu.VMEM((CPT, 128), jnp.int32),
            pltpu.VMEM((NPAD,), jnp.float32),
            pltpu.VMEM((NPAD,), jnp.float32),
            pltpu.VMEM((L,), jnp.float32),
        ],
    )(asrc, adst, srcp32, dstp32)


def _sc_gat(h2_split, asrc, adst, srcp_off, dstp, maxes):
    return pl.kernel(
        _sc_gat_body,
        compiler_params=_SC_PARAMS,
        out_type=(jax.ShapeDtypeStruct((NC, NPAD, 64), jnp.float32),
                  jax.ShapeDtypeStruct((NC, NS, NPAD), jnp.float32)),
        mesh=_MESH,
        scratch_types=[
            pltpu.VMEM((CPT2, 128), jnp.int32),
            pltpu.VMEM((CPT2, 128), jnp.int32),
            pltpu.VMEM((NPAD,), jnp.float32),
            pltpu.VMEM((NPAD,), jnp.float32),
            pltpu.VMEM((128, 64), jnp.float32),
            pltpu.VMEM((NBLK, 64), jnp.float32),
            pltpu.VMEM((128,), jnp.float32),
            pltpu.VMEM((NPAD,), jnp.float32),
            pltpu.VMEM((NW, L), jnp.float32),
            pltpu.VMEM_SHARED((NPAD, 64), jnp.float32),
            pltpu.SemaphoreType.DMA,
        ],
    )(h2_split, asrc, adst, srcp_off, dstp, maxes)


def _tc_a_body(x_ref, w1_ref, degp_ref, hs_ref, dinv_ref):
    deg = jnp.sum(degp_ref[...], axis=0) + 1.0
    dinv = lax.rsqrt(deg)
    h = jnp.dot(x_ref[...], w1_ref[...], preferred_element_type=jnp.float32)
    hs_ref[...] = h * dinv[None, :].reshape(128, 1)
    dinv_ref[...] = dinv[None, None, :]


def _tc_a(x_pad, W1, deg_parts):
    return pl.pallas_call(
        _tc_a_body,
        grid=(NBLK,),
        in_specs=[
            pl.BlockSpec((128, 192), lambda i: (i, 0)),
            pl.BlockSpec((192, 128), lambda i: (0, 0)),
            pl.BlockSpec((NW, 128), lambda i: (0, i)),
        ],
        out_specs=[
            pl.BlockSpec((128, 128), lambda i: (i, 0)),
            pl.BlockSpec((1, 1, 128), lambda i: (i, 0, 0)),
        ],
        out_shape=[
            jax.ShapeDtypeStruct((NPAD, 128), jnp.float32),
            jax.ShapeDtypeStruct((NBLK, 1, 128), jnp.float32),
        ],
    )(x_pad, W1, deg_parts)


def _tc_b_body(s1p_ref, hs1_ref, dinv_ref, b1_ref, wg_ref, asv_ref, adv_ref,
               h2_ref, asrc_ref, adst_ref):
    dv = dinv_ref[...].reshape(128, 1)
    S = jnp.concatenate([s1p_ref[0], s1p_ref[1]], axis=1)
    g1 = dv * (S + hs1_ref[...]) + b1_ref[...]
    h2 = jnp.dot(g1, wg_ref[...], preferred_element_type=jnp.float32)
    h2_ref[...] = h2
    asrc_ref[...] = jnp.sum(h2 * asv_ref[...], axis=1)[None, None, :]
    adst_ref[...] = jnp.sum(h2 * adv_ref[...], axis=1)[None, None, :]


def _tc_b(S1p, hs1, dinv3d, b1, Wg, att_src, att_dst):
    return pl.pallas_call(
        _tc_b_body,
        grid=(NBLK,),
        in_specs=[
            pl.BlockSpec((2, 128, 64), lambda i: (0, i, 0)),
            pl.BlockSpec((128, 128), lambda i: (i, 0)),
            pl.BlockSpec((1, 1, 128), lambda i: (i, 0, 0)),
            pl.BlockSpec((1, 128), lambda i: (0, 0)),
            pl.BlockSpec((128, 128), lambda i: (0, 0)),
            pl.BlockSpec((1, 128), lambda i: (0, 0)),
            pl.BlockSpec((1, 128), lambda i: (0, 0)),
        ],
        out_specs=[
            pl.BlockSpec((128, 128), lambda i: (i, 0)),
            pl.BlockSpec((1, 1, 128), lambda i: (i, 0, 0)),
            pl.BlockSpec((1, 1, 128), lambda i: (i, 0, 0)),
        ],
        out_shape=[
            jax.ShapeDtypeStruct((NPAD, 128), jnp.float32),
            jax.ShapeDtypeStruct((NBLK, 1, 128), jnp.float32),
            jax.ShapeDtypeStruct((NBLK, 1, 128), jnp.float32),
        ],
    )(S1p, hs1, dinv3d, b1, Wg, att_src, att_dst)


def _tc_c_body(s2p_ref, denp_ref, h2_ref, asrc_ref, adst_ref, maxes_ref,
               dinv_ref, bg_ref, w2_ref, hs3_ref):
    C = jnp.max(maxes_ref[...])
    z = asrc_ref[...] + adst_ref[...]
    es = jnp.exp(jnp.maximum(z, 0.2 * z) - C).reshape(128, 1)
    Stot = jnp.concatenate([s2p_ref[0], s2p_ref[1]], axis=1) + es * h2_ref[...]
    den = jnp.sum(denp_ref[0], axis=0)[None, :].reshape(128, 1) + es
    out2 = jnp.maximum(Stot / (den + 1e-16) + bg_ref[...], 0.0)
    h3 = jnp.dot(out2, w2_ref[...], preferred_element_type=jnp.float32)
    hs3_ref[...] = h3 * dinv_ref[...].reshape(128, 1)


def _tc_c(S2p, den_parts, h2, asrc3d, adst3d, maxes, dinv3d, bg, W2):
    return pl.pallas_call(
        _tc_c_body,
        grid=(NBLK,),
        in_specs=[
            pl.BlockSpec((2, 128, 64), lambda i: (0, i, 0)),
            pl.BlockSpec((1, NS, 128), lambda i: (0, 0, i)),
            pl.BlockSpec((128, 128), lambda i: (i, 0)),
            pl.BlockSpec((1, 1, 128), lambda i: (i, 0, 0)),
            pl.BlockSpec((1, 1, 128), lambda i: (i, 0, 0)),
            pl.BlockSpec((NW, L), lambda i: (0, 0)),
            pl.BlockSpec((1, 1, 128), lambda i: (i, 0, 0)),
            pl.BlockSpec((1, 128), lambda i: (0, 0)),
            pl.BlockSpec((128, 64), lambda i: (0, 0)),
        ],
        out_specs=[pl.BlockSpec((128, 64), lambda i: (i, 0))],
        out_shape=[jax.ShapeDtypeStruct((NPAD, 64), jnp.float32)],
    )(S2p, den_parts, h2, asrc3d, adst3d, maxes, dinv3d, bg, W2)


def _tc_d_body(s3p_ref, hs3_ref, dinv_ref, b2_ref, wf_ref, bf_ref, out_ref):
    dv = dinv_ref[...].reshape(128, 1)
    S = jnp.concatenate([s3p_ref[0], s3p_ref[1]], axis=1)
    g3 = dv * (S + hs3_ref[...]) + b2_ref[...]
    out_ref[...] = (jnp.dot(g3, wf_ref[...], preferred_element_type=jnp.float32)
                    + bf_ref[...])


def _tc_d(S3p, hs3, dinv3d, b2, Wf, bf):
    return pl.pallas_call(
        _tc_d_body,
        grid=(NBLK,),
        in_specs=[
            pl.BlockSpec((2, 128, 32), lambda i: (0, i, 0)),
            pl.BlockSpec((128, 64), lambda i: (i, 0)),
            pl.BlockSpec((1, 1, 128), lambda i: (i, 0, 0)),
            pl.BlockSpec((1, 64), lambda i: (0, 0)),
            pl.BlockSpec((64, 192), lambda i: (0, 0)),
            pl.BlockSpec((1, 192), lambda i: (0, 0)),
        ],
        out_specs=[pl.BlockSpec((128, 192), lambda i: (i, 0))],
        out_shape=[jax.ShapeDtypeStruct((NPAD, 192), jnp.float32)],
    )(S3p, hs3, dinv3d, b2, Wf, bf)


def _split_rows(h, F2):
    return jnp.concatenate([h[:, :F2], h[:, F2:]], axis=0)


def kernel(x, edge_index, W1, b1, Wg, att_src, att_dst, bg, W2, b2, Wf, bf):
    src = edge_index[0].astype(jnp.int32)
    dst = edge_index[1].astype(jnp.int32)
    pad32 = ((0, 0), (0, NPAD - EPT))
    srcp32 = jnp.pad(src.reshape(NW, EPT), pad32, constant_values=N).reshape(NW, CPT, 128)
    dstp32 = jnp.pad(dst.reshape(NW, EPT), pad32, constant_values=N).reshape(NW, CPT, 128)
    pad16 = ((0, 0), (0, EPAD2 - EPT2))
    srcp16 = jnp.pad(src.reshape(NS, EPT2), pad16, constant_values=N).reshape(NS, CPT2, 128)
    dstp16 = jnp.pad(dst.reshape(NS, EPT2), pad16, constant_values=N).reshape(NS, CPT2, 128)
    srcp_off = jnp.stack([srcp16, srcp16 + NPAD], axis=0)
    x_pad = jnp.pad(x, ((0, NPAD - N), (0, 0)))

    deg_parts = _sc_deg(dstp32)
    hs1, dinv3d = _tc_a(x_pad, W1, deg_parts)
    S1p = _sc_gcn(_split_rows(hs1, 64), srcp_off, dstp16, 64)
    h2, asrc3d, adst3d = _tc_b(S1p, hs1, dinv3d, b1.reshape(1, 128), Wg,
                               att_src.reshape(1, 128), att_dst.reshape(1, 128))
    asrc = asrc3d.reshape(NPAD)
    adst = adst3d.reshape(NPAD)
    maxes = _sc_amax(asrc, adst, srcp32, dstp32)
    S2p, den_parts = _sc_gat(_split_rows(h2, 64), asrc, adst, srcp_off,
                             dstp16, maxes)
    hs3, = _tc_c(S2p, den_parts, h2, asrc3d, adst3d, maxes, dinv3d,
                 bg.reshape(1, 128), W2)
    S3p = _sc_gcn(_split_rows(hs3, 32), srcp_off, dstp16, 32)
    out, = _tc_d(S3p, hs3, dinv3d, b2.reshape(1, 64), Wf, bf.reshape(1, 192))
    return out[:N]
